# Initial kernel scaffold; baseline (speedup 1.0000x reference)
import jax
import jax.numpy as jnp
from jax.experimental import pallas as pl


def kernel(z, codebook):
    raise NotImplementedError("write your pallas kernel here")


# speedup vs baseline: 1.1694x; 1.1694x over previous
import functools

import jax
import jax.numpy as jnp
from jax import lax
from jax.experimental import pallas as pl
from jax.experimental.pallas import tpu as pltpu
from jax.experimental.pallas import tpu_sc as plsc

NUM_CODES = 1024
D = 64
BETA = 0.25

B0 = 64
B1 = 576
N_ROWS = B0 * B1

NC = 2
NS = 16
NW = NC * NS
B_PER_W = N_ROWS // NW


def _dist_argmin_kernel(z_ref, c_ref, idx_ref, loss_ref):
    i = pl.program_id(0)
    z = z_ref[0]
    c = c_ref[...]
    zc = lax.dot_general(z, c, (((1,), (1,)), ((), ())),
                         preferred_element_type=jnp.float32)
    z_sq = jnp.sum(z * z, axis=1, keepdims=True)
    c_sq = jnp.sum(c * c, axis=1)[None, :]
    dist = (z_sq - 2.0 * zc) + c_sq
    m = jnp.min(dist, axis=1, keepdims=True)
    codes = lax.broadcasted_iota(jnp.int32, dist.shape, 1)
    idx = jnp.min(jnp.where(dist == m, codes, NUM_CODES), axis=1)
    idx_ref[0, 0, :] = idx

    @pl.when(i == 0)
    def _():
        loss_ref[...] = jnp.zeros((1, 1), jnp.float32)

    loss_ref[...] += jnp.sum(m).reshape(1, 1)


def _tc_dist_argmin(z, codebook):
    z3 = z.reshape(B0, B1, D)
    idx, loss_sum = pl.pallas_call(
        _dist_argmin_kernel,
        grid=(B0,),
        in_specs=[
            pl.BlockSpec((1, B1, D), lambda i: (i, 0, 0)),
            pl.BlockSpec((NUM_CODES, D), lambda i: (0, 0)),
        ],
        out_specs=[
            pl.BlockSpec((1, 1, B1), lambda i: (i, 0, 0)),
            pl.BlockSpec((1, 1), lambda i: (0, 0)),
        ],
        out_shape=[
            jax.ShapeDtypeStruct((B0, 1, B1), jnp.int32),
            jax.ShapeDtypeStruct((1, 1), jnp.float32),
        ],
    )(z3, codebook)
    return idx.reshape(N_ROWS), loss_sum


DPAD = 128
CHUNK = 576


def _sc_gather(codebook_padded, idx_flat):
    mesh = plsc.VectorSubcoreMesh(core_axis_name="c", subcore_axis_name="s")

    @functools.partial(
        pl.kernel,
        mesh=mesh,
        out_type=jax.ShapeDtypeStruct((N_ROWS, DPAD), jnp.float32),
        scratch_types=[
            pltpu.VMEM((CHUNK,), jnp.int32),
            pltpu.VMEM((CHUNK, DPAD), jnp.float32),
            pltpu.SemaphoreType.DMA,
        ],
    )
    def k(table_hbm, idx_hbm, out_hbm, idx_v, rows_v, sem):
        wid = lax.axis_index("s") * NC + lax.axis_index("c")
        base = wid * B_PER_W

        @pl.loop(0, B_PER_W // CHUNK)
        def _(j):
            b = base + j * CHUNK
            pltpu.sync_copy(idx_hbm.at[pl.ds(b, CHUNK)], idx_v)
            pltpu.async_copy(table_hbm.at[idx_v], rows_v, sem).wait()
            pltpu.sync_copy(rows_v, out_hbm.at[pl.ds(b, CHUNK)])

    return k(codebook_padded, idx_flat)


def kernel(z, codebook):
    idx, loss_sum = _tc_dist_argmin(z, codebook)
    codebook_padded = jnp.pad(codebook, ((0, 0), (0, DPAD - D)))
    z_q = _sc_gather(codebook_padded, idx)[:, :D]
    loss = loss_sum[0, 0] * (2.0 * BETA / (N_ROWS * D))
    return z_q.reshape(z.shape), loss


# speedup vs baseline: 1.4511x; 1.2408x over previous
import functools

import jax
import jax.numpy as jnp
from jax import lax
from jax.experimental import pallas as pl
from jax.experimental.pallas import tpu as pltpu
from jax.experimental.pallas import tpu_sc as plsc

NUM_CODES = 1024
D = 64
BETA = 0.25

B0 = 64
B1 = 576
N_ROWS = B0 * B1

NC = 2
NS = 16
NW = NC * NS
B_PER_W = N_ROWS // NW


def _dist_argmin_kernel(z_ref, cm2_ref, idx_ref, loss_ref, csq_ref):
    i = pl.program_id(0)
    cm2 = cm2_ref[...]

    @pl.when(i == 0)
    def _():
        csq_ref[...] = jnp.sum(cm2 * cm2, axis=1, keepdims=True) * 0.25
        loss_ref[...] = jnp.zeros((1, 1), jnp.float32)

    z = z_ref[0]
    m2zc = lax.dot_general(cm2, z, (((1,), (1,)), ((), ())),
                           preferred_element_type=jnp.float32)
    z_sq = jnp.sum(z * z, axis=1, keepdims=True).T
    dist = (z_sq + m2zc) + csq_ref[...]
    m = jnp.min(dist, axis=0, keepdims=True)
    codes = lax.broadcasted_iota(jnp.int32, dist.shape, 0)
    idx = jnp.min(jnp.where(dist == m, codes, NUM_CODES), axis=0)
    idx_ref[0, 0, :] = idx
    loss_ref[...] += jnp.sum(m).reshape(1, 1)


def _tc_dist_argmin(z, codebook_m2):
    z3 = z.reshape(B0, B1, D)
    idx, loss_sum = pl.pallas_call(
        _dist_argmin_kernel,
        grid=(B0,),
        in_specs=[
            pl.BlockSpec((1, B1, D), lambda i: (i, 0, 0)),
            pl.BlockSpec((NUM_CODES, D), lambda i: (0, 0)),
        ],
        out_specs=[
            pl.BlockSpec((1, 1, B1), lambda i: (i, 0, 0)),
            pl.BlockSpec((1, 1), lambda i: (0, 0)),
        ],
        out_shape=[
            jax.ShapeDtypeStruct((B0, 1, B1), jnp.int32),
            jax.ShapeDtypeStruct((1, 1), jnp.float32),
        ],
        scratch_shapes=[pltpu.VMEM((NUM_CODES, 1), jnp.float32)],
    )(z3, codebook_m2)
    return idx.reshape(N_ROWS), loss_sum


DPAD = 128
CHUNK = 576


def _sc_gather(codebook_padded, idx_flat):
    mesh = plsc.VectorSubcoreMesh(core_axis_name="c", subcore_axis_name="s")

    @functools.partial(
        pl.kernel,
        mesh=mesh,
        out_type=jax.ShapeDtypeStruct((N_ROWS, DPAD), jnp.float32),
        scratch_types=[
            pltpu.VMEM((CHUNK,), jnp.int32),
            pltpu.VMEM((CHUNK, DPAD), jnp.float32),
            pltpu.SemaphoreType.DMA,
        ],
    )
    def k(table_hbm, idx_hbm, out_hbm, idx_v, rows_v, sem):
        wid = lax.axis_index("s") * NC + lax.axis_index("c")
        base = wid * B_PER_W

        @pl.loop(0, B_PER_W // CHUNK)
        def _(j):
            b = base + j * CHUNK
            pltpu.sync_copy(idx_hbm.at[pl.ds(b, CHUNK)], idx_v)
            pltpu.async_copy(table_hbm.at[idx_v], rows_v, sem).wait()
            pltpu.sync_copy(rows_v, out_hbm.at[pl.ds(b, CHUNK)])

    return k(codebook_padded, idx_flat)


def kernel(z, codebook):
    idx, loss_sum = _tc_dist_argmin(z, codebook * -2.0)
    codebook_padded = jnp.pad(codebook, ((0, 0), (0, DPAD - D)))
    z_q = _sc_gather(codebook_padded, idx)[:, :D]
    loss = loss_sum[0, 0] * (2.0 * BETA / (N_ROWS * D))
    return z_q.reshape(z.shape), loss


# speedup vs baseline: 1.4759x; 1.0171x over previous
import functools

import jax
import jax.numpy as jnp
from jax import lax
from jax.experimental import pallas as pl
from jax.experimental.pallas import tpu as pltpu
from jax.experimental.pallas import tpu_sc as plsc

NUM_CODES = 1024
D = 64
BETA = 0.25

B0 = 64
B1 = 576
N_ROWS = B0 * B1

NC = 2
NS = 16
NW = NC * NS
B_PER_W = N_ROWS // NW


def _dist_argmin_kernel(z_ref, cm2_ref, idx_ref, loss_ref, csq_ref):
    i = pl.program_id(0)
    cm2 = cm2_ref[...]

    @pl.when(i == 0)
    def _():
        csq_ref[...] = jnp.sum(cm2 * cm2, axis=1, keepdims=True) * 0.25
        loss_ref[...] = jnp.zeros((1, 1), jnp.float32)

    z = z_ref[...]
    m2zc = lax.dot_general(cm2, z, (((1,), (1,)), ((), ())),
                           preferred_element_type=jnp.float32)
    z_sq = jnp.sum(z * z, axis=1, keepdims=True).T
    dist = (z_sq + m2zc) + csq_ref[...]
    m = jnp.min(dist, axis=0, keepdims=True)
    codes = lax.broadcasted_iota(jnp.int32, dist.shape, 0)
    idx = jnp.min(jnp.where(dist == m, codes, NUM_CODES), axis=0)
    idx_ref[0, 0, :] = idx
    loss_ref[...] += jnp.sum(m).reshape(1, 1)


def _tc_dist_argmin(z, codebook_m2):
    z2 = z.reshape(N_ROWS, D)
    idx, loss_sum = pl.pallas_call(
        _dist_argmin_kernel,
        grid=(B0,),
        in_specs=[
            pl.BlockSpec((B1, D), lambda i: (i, 0)),
            pl.BlockSpec((NUM_CODES, D), lambda i: (0, 0)),
        ],
        out_specs=[
            pl.BlockSpec((1, 1, B1), lambda i: (i, 0, 0)),
            pl.BlockSpec((1, 1), lambda i: (0, 0)),
        ],
        out_shape=[
            jax.ShapeDtypeStruct((B0, 1, B1), jnp.int32),
            jax.ShapeDtypeStruct((1, 1), jnp.float32),
        ],
        scratch_shapes=[pltpu.VMEM((NUM_CODES, 1), jnp.float32)],
    )(z2, codebook_m2)
    return idx.reshape(N_ROWS), loss_sum


DPAD = 128
CHUNK = 384


def _sc_gather(codebook_padded, idx_flat):
    mesh = plsc.VectorSubcoreMesh(core_axis_name="c", subcore_axis_name="s")

    @functools.partial(
        pl.kernel,
        mesh=mesh,
        out_type=jax.ShapeDtypeStruct((N_ROWS, DPAD), jnp.float32),
        scratch_types=[
            pltpu.VMEM((CHUNK,), jnp.int32),
            pltpu.VMEM((CHUNK,), jnp.int32),
            pltpu.VMEM((CHUNK, DPAD), jnp.float32),
            pltpu.VMEM((CHUNK, DPAD), jnp.float32),
            pltpu.SemaphoreType.DMA,
            pltpu.SemaphoreType.DMA,
            pltpu.SemaphoreType.DMA,
            pltpu.SemaphoreType.DMA,
        ],
    )
    def k(table_hbm, idx_hbm, out_hbm, iv0, iv1, rv0, rv1,
          gsem0, gsem1, osem0, osem1):
        wid = lax.axis_index("s") * NC + lax.axis_index("c")
        base = wid * B_PER_W
        pltpu.sync_copy(idx_hbm.at[pl.ds(base, CHUNK)], iv0)
        g0 = pltpu.async_copy(table_hbm.at[iv0], rv0, gsem0)
        pltpu.sync_copy(idx_hbm.at[pl.ds(base + CHUNK, CHUNK)], iv1)
        g1 = pltpu.async_copy(table_hbm.at[iv1], rv1, gsem1)
        g0.wait()
        o0 = pltpu.async_copy(rv0, out_hbm.at[pl.ds(base, CHUNK)], osem0)
        pltpu.sync_copy(idx_hbm.at[pl.ds(base + 2 * CHUNK, CHUNK)], iv0)
        g1.wait()
        o1 = pltpu.async_copy(rv1, out_hbm.at[pl.ds(base + CHUNK, CHUNK)], osem1)
        o0.wait()
        g2 = pltpu.async_copy(table_hbm.at[iv0], rv0, gsem0)
        g2.wait()
        o1.wait()
        pltpu.sync_copy(rv0, out_hbm.at[pl.ds(base + 2 * CHUNK, CHUNK)])

    return k(codebook_padded, idx_flat)


def kernel(z, codebook):
    idx, loss_sum = _tc_dist_argmin(z, codebook * -2.0)
    codebook_padded = jnp.pad(codebook, ((0, 0), (0, DPAD - D)))
    z_q = _sc_gather(codebook_padded, idx)[:, :D]
    loss = loss_sum[0, 0] * (2.0 * BETA / (N_ROWS * D))
    return z_q.reshape(z.shape), loss
